# baseline (speedup 1.0000x reference)
import functools

import jax
import jax.numpy as jnp
from jax import lax
from jax.experimental import pallas as pl
from jax.experimental.pallas import tpu as pltpu
from jax.experimental.pallas import tpu_sc as plsc

D = 64
B, T, N = 8, 64, 1024

NC, NS = 2, 16
NW = NC * NS
WB = NW // B
NCHUNK = N // WB
NJ = D // 16


def _sc_gather_kernel(regions_hbm, eids_hbm, bsp_hbm, remb_hbm, eemb_hbm,
                      reg_out_hbm, cb_out_hbm, idx_v, reg_v, eids_v,
                      eid_rows_v, bsp_v, sem):
    c = lax.axis_index("c")
    s = lax.axis_index("s")
    wid = s * NC + c
    b = wid // WB
    nbase = (wid % WB) * NCHUNK

    pltpu.sync_copy(regions_hbm.at[b, pl.ds(nbase, NCHUNK)], idx_v)
    pltpu.async_copy(remb_hbm.at[idx_v], reg_v, sem).wait()
    pltpu.sync_copy(reg_v, reg_out_hbm.at[b, pl.ds(nbase, NCHUNK)])

    @pl.when(wid == 0)
    def _():
        pltpu.sync_copy(eids_hbm, eids_v)
        pltpu.sync_copy(bsp_hbm, bsp_v)
        pltpu.async_copy(eemb_hbm.at[eids_v], eid_rows_v, sem).wait()
        for bb in range(B):
            for j in range(NJ):
                sl = pl.ds(16 * j, 16)
                eid_rows_v[bb, sl] = eid_rows_v[bb, sl] + bsp_v[sl]
        pltpu.sync_copy(eid_rows_v, cb_out_hbm)


@functools.lru_cache(maxsize=1)
def _sc_gather():
    return pl.kernel(
        _sc_gather_kernel,
        out_type=(jax.ShapeDtypeStruct((B, N, D), jnp.float32),
                  jax.ShapeDtypeStruct((B, D), jnp.float32)),
        mesh=plsc.VectorSubcoreMesh(core_axis_name="c", subcore_axis_name="s",
                                    num_cores=NC, num_subcores=NS),
        scratch_types=[
            pltpu.VMEM((NCHUNK,), jnp.int32),
            pltpu.VMEM((NCHUNK, D), jnp.float32),
            pltpu.VMEM((B,), jnp.int32),
            pltpu.VMEM((B, D), jnp.float32),
            pltpu.VMEM((D,), jnp.float32),
            pltpu.SemaphoreType.DMA,
        ],
        compiler_params=pltpu.CompilerParams(use_tc_tiling_on_sc=False),
    )


TT = 64


def _tc_body(s_ref, w_ref, slot_ref, reg_ref, cbt_ref, o_ref, baset_s):
    @pl.when(pl.program_id(1) == 0)
    def _():
        cbt = cbt_ref[...]
        bsel = jax.lax.broadcasted_iota(jnp.int32, (D, B), 1) == pl.program_id(0)
        cb_col = jnp.sum(jnp.where(bsel, cbt, 0.0), axis=1, keepdims=True)
        baset_s[...] = (jnp.transpose(reg_ref[0], (1, 0))
                        + jnp.transpose(slot_ref[...], (1, 0))
                        + cb_col)

    s = s_ref[0]
    w = w_ref[...]
    o_ref[0] = s[:, None, :] * w[None, :, :] + baset_s[...][None, :, :]


def _tc_broadcast(spikes, w_spike, slot, reg, cbt):
    outp = pl.pallas_call(
        _tc_body,
        grid=(B, T // TT),
        in_specs=[
            pl.BlockSpec((1, TT, N), lambda i, j: (i, j, 0)),
            pl.BlockSpec((D, 1), lambda i, j: (0, 0)),
            pl.BlockSpec((N, D), lambda i, j: (0, 0)),
            pl.BlockSpec((1, N, D), lambda i, j: (i, 0, 0)),
            pl.BlockSpec((D, B), lambda i, j: (0, 0)),
        ],
        out_specs=pl.BlockSpec((1, TT, D, N), lambda i, j: (i, j, 0, 0)),
        out_shape=jax.ShapeDtypeStruct((B, T, D, N), jnp.float32),
        scratch_shapes=[pltpu.VMEM((D, N), jnp.float32)],
    )(spikes, w_spike, slot, reg, cbt)
    return jnp.swapaxes(outp, 2, 3)


def kernel(spikes, neuron_regions, eids, w_spike, b_spike, neuron_slot,
           region_emb, eid_emb):
    reg, cb = _sc_gather()(neuron_regions.astype(jnp.int32),
                           eids.astype(jnp.int32), b_spike,
                           region_emb, eid_emb)
    return _tc_broadcast(spikes, w_spike, neuron_slot, reg, cb.T)


# speedup vs baseline: 1.0570x; 1.0570x over previous
import functools

import jax
import jax.numpy as jnp
from jax import lax
from jax.experimental import pallas as pl
from jax.experimental.pallas import tpu as pltpu
from jax.experimental.pallas import tpu_sc as plsc

D = 64
B, T, N = 8, 64, 1024

NC, NS = 2, 16
NW = NC * NS
WB = NW // B
NCHUNK = N // WB
NJ = D // 16


def _sc_gather_kernel(regions_hbm, eids_hbm, bsp_hbm, remb_hbm, eemb_hbm,
                      reg_out_hbm, cb_out_hbm, idx_v, reg_v, eids_v,
                      eid_rows_v, bsp_v, sem):
    c = lax.axis_index("c")
    s = lax.axis_index("s")
    wid = s * NC + c
    b = wid // WB
    nbase = (wid % WB) * NCHUNK

    for k in range(NCHUNK // 128):
        pltpu.sync_copy(regions_hbm.at[b, pl.ds(nbase + 128 * k, 128)],
                        idx_v.at[k])
    cps = [pltpu.async_copy(remb_hbm.at[idx_v.at[k]],
                            reg_v.at[pl.ds(128 * k, 128)], sem)
           for k in range(NCHUNK // 128)]
    for cp in cps:
        cp.wait()
    pltpu.sync_copy(reg_v, reg_out_hbm.at[b, pl.ds(nbase, NCHUNK)])

    @pl.when(wid == 0)
    def _():
        pltpu.sync_copy(eids_hbm, eids_v)
        pltpu.sync_copy(bsp_hbm, bsp_v)
        pltpu.async_copy(eemb_hbm.at[eids_v], eid_rows_v, sem).wait()
        for bb in range(B):
            for j in range(NJ):
                sl = pl.ds(16 * j, 16)
                eid_rows_v[bb, sl] = eid_rows_v[bb, sl] + bsp_v[sl]
        pltpu.sync_copy(eid_rows_v, cb_out_hbm)


@functools.lru_cache(maxsize=1)
def _sc_gather():
    return pl.kernel(
        _sc_gather_kernel,
        out_type=(jax.ShapeDtypeStruct((B, N, 2 * D), jnp.float32),
                  jax.ShapeDtypeStruct((B, 2 * D), jnp.float32)),
        mesh=plsc.VectorSubcoreMesh(core_axis_name="c", subcore_axis_name="s",
                                    num_cores=NC, num_subcores=NS),
        scratch_types=[
            pltpu.VMEM((NCHUNK // 128, 128), jnp.int32),
            pltpu.VMEM((NCHUNK, 2 * D), jnp.float32),
            pltpu.VMEM((B,), jnp.int32),
            pltpu.VMEM((B, 2 * D), jnp.float32),
            pltpu.VMEM((D,), jnp.float32),
            pltpu.SemaphoreType.DMA,
        ],
        compiler_params=pltpu.CompilerParams(use_tc_tiling_on_sc=True),
    )


TT = 64


def _tc_body(s_ref, w_ref, slot_ref, reg_ref, cbt_ref, o_ref, baset_s):
    @pl.when(pl.program_id(1) == 0)
    def _():
        cbt = cbt_ref[...]
        bsel = jax.lax.broadcasted_iota(jnp.int32, (D, B), 1) == pl.program_id(0)
        cb_col = jnp.sum(jnp.where(bsel, cbt, 0.0), axis=1, keepdims=True)
        baset_s[...] = (jnp.transpose(reg_ref[0][:, :D], (1, 0))
                        + jnp.transpose(slot_ref[...], (1, 0))
                        + cb_col)

    s = s_ref[0]
    w = w_ref[...]
    o_ref[0] = s[:, None, :] * w[None, :, :] + baset_s[...][None, :, :]


def _tc_broadcast(spikes, w_spike, slot, reg, cbt):
    outp = pl.pallas_call(
        _tc_body,
        grid=(B, T // TT),
        in_specs=[
            pl.BlockSpec((1, TT, N), lambda i, j: (i, j, 0)),
            pl.BlockSpec((D, 1), lambda i, j: (0, 0)),
            pl.BlockSpec((N, D), lambda i, j: (0, 0)),
            pl.BlockSpec((1, N, 2 * D), lambda i, j: (i, 0, 0)),
            pl.BlockSpec((D, B), lambda i, j: (0, 0)),
        ],
        out_specs=pl.BlockSpec((1, TT, D, N), lambda i, j: (i, j, 0, 0)),
        out_shape=jax.ShapeDtypeStruct((B, T, D, N), jnp.float32),
        scratch_shapes=[pltpu.VMEM((D, N), jnp.float32)],
    )(spikes, w_spike, slot, reg, cbt)
    return jnp.swapaxes(outp, 2, 3)


def kernel(spikes, neuron_regions, eids, w_spike, b_spike, neuron_slot,
           region_emb, eid_emb):
    remb_p = jnp.pad(region_emb, ((0, 0), (0, D)))
    eemb_p = jnp.pad(eid_emb, ((0, 0), (0, D)))
    reg, cb = _sc_gather()(neuron_regions.astype(jnp.int32),
                           eids.astype(jnp.int32), b_spike, remb_p, eemb_p)
    return _tc_broadcast(spikes, w_spike, neuron_slot, reg, cb[:, :D].T)
